# baseline (speedup 1.0000x reference)
import functools

import jax
import jax.numpy as jnp
from jax import lax
from jax.experimental import pallas as pl
from jax.experimental.pallas import tpu as pltpu
from jax.experimental.pallas import tpu_sc as plsc

_D = 768
_NPOS = 12
_NNEG = 12
_DEEP = 4
_PROJ = 896
_LSEQ = 77
_NSEG = 4
_CHUNK = 16

_N_CTX_SEG = (_NPOS, _NPOS + _NNEG, _NPOS, _NPOS + _NNEG)

_TOK_WORKERS = tuple(
    (s, j0)
    for s in range(_NSEG)
    for j0 in range(0, _LSEQ - _N_CTX_SEG[s], _CHUNK)
)
_CTX_WORKERS = (
    (0, 0, 1), (0, 1, 1), (1, 1, 1 + _NPOS),
    (2, 2, 1), (2, 3, 1), (3, 3, 1 + _NPOS),
)

_info = plsc.get_sparse_core_info()
_NC = _info.num_cores


@functools.partial(
    pl.kernel,
    mesh=plsc.VectorSubcoreMesh(core_axis_name="c", subcore_axis_name="s"),
    out_type=tuple(
        jax.ShapeDtypeStruct((_LSEQ, _D), jnp.float32) for _ in range(_NSEG)
    ),
    scratch_types=[
        pltpu.VMEM((_NSEG * _LSEQ,), jnp.int32),
        pltpu.VMEM((_CHUNK, _D), jnp.float32),
        pltpu.SemaphoreType.DMA,
    ],
    compiler_params=pltpu.CompilerParams(needs_layout_passes=False),
)
def _sc_prompts(table, gpos, gneg, lpos, lneg, tok,
                out_gp, out_gn, out_lp, out_ln, tokv, buf, sem):
    ctx_refs = (gpos, gneg, lpos, lneg)
    out_refs = (out_gp, out_gn, out_lp, out_ln)
    wid = lax.axis_index("s") * _NC + lax.axis_index("c")
    iota = lax.iota(jnp.int32, _CHUNK)
    b = 0
    for s, j0 in _TOK_WORKERS:
        nctx = _N_CTX_SEG[s]
        njobs = _LSEQ - nctx

        @pl.when(wid == b)
        def _(s=s, j0=j0, nctx=nctx, njobs=njobs):
            pltpu.sync_copy(tok, tokv)
            i = jnp.minimum(iota + j0, njobs - 1)
            pos = jnp.where(i < 1, 0, nctx + i)
            sidx = plsc.load_gather(tokv, [_LSEQ * s + pos])
            pltpu.async_copy(table.at[sidx], buf, sem).wait()
            pltpu.async_copy(buf, out_refs[s].at[pos], sem).wait()

        b += 1
    for src, s, dst0 in _CTX_WORKERS:
        @pl.when(wid == b)
        def _(src=src, s=s, dst0=dst0):
            k = jnp.minimum(iota, _NPOS - 1)
            pltpu.async_copy(ctx_refs[src].at[k], buf, sem).wait()
            pltpu.async_copy(buf, out_refs[s].at[dst0 + k], sem).wait()

        b += 1


def _proj_body(cp_ref, w_ref, b_ref, out_ref):
    out_ref[...] = (
        jnp.dot(cp_ref[0], w_ref[0], preferred_element_type=jnp.float32)
        + b_ref[0]
    )[None]


_proj = pl.pallas_call(
    _proj_body,
    grid=(8,),
    in_specs=[
        pl.BlockSpec((1, _DEEP, _D), lambda l: (l, 0, 0)),
        pl.BlockSpec((1, _D, _PROJ), lambda l: (l, 0, 0)),
        pl.BlockSpec((1, 1, _PROJ), lambda l: (l, 0, 0)),
    ],
    out_specs=pl.BlockSpec((1, _DEEP, _PROJ), lambda l: (l, 0, 0)),
    out_shape=jax.ShapeDtypeStruct((8, _DEEP, _PROJ), jnp.float32),
)


def kernel(token_embedding, ctx_global_pos, ctx_global_neg, ctx_local_pos,
           ctx_local_neg, compound_prompts_text, proj_W, proj_b,
           tokens_global_pos, tokens_global_neg, tokens_local_pos,
           tokens_local_neg):
    tok_flat = jnp.concatenate([
        tokens_global_pos.reshape(-1), tokens_global_neg.reshape(-1),
        tokens_local_pos.reshape(-1), tokens_local_neg.reshape(-1),
    ])
    outs = _sc_prompts(
        token_embedding,
        ctx_global_pos.reshape(_NPOS, _D), ctx_global_neg.reshape(_NNEG, _D),
        ctx_local_pos.reshape(_NPOS, _D), ctx_local_neg.reshape(_NNEG, _D),
        tok_flat,
    )
    projected = _proj(compound_prompts_text, proj_W,
                      proj_b.reshape(8, 1, _PROJ))
    return (*(o.reshape(1, _LSEQ, _D) for o in outs), projected)


# speedup vs baseline: 1.0049x; 1.0049x over previous
import functools

import jax
import jax.numpy as jnp
from jax import lax
from jax.experimental import pallas as pl
from jax.experimental.pallas import tpu as pltpu
from jax.experimental.pallas import tpu_sc as plsc

_D = 768
_NPOS = 12
_DEEP = 4
_PROJ = 896
_LSEQ = 77
_NSEG = 4
_CHUNK = 16

_N_CTX_SEG = (_NPOS, 2 * _NPOS, _NPOS, 2 * _NPOS)
_WPS = 6

_info = plsc.get_sparse_core_info()
_NC = _info.num_cores


@functools.partial(
    pl.kernel,
    mesh=plsc.VectorSubcoreMesh(core_axis_name="c", subcore_axis_name="s"),
    out_type=tuple(
        jax.ShapeDtypeStruct((_LSEQ, _D), jnp.float32) for _ in range(_NSEG)
    ),
    scratch_types=[
        pltpu.VMEM((_NSEG * _LSEQ,), jnp.int32),
        pltpu.VMEM((_CHUNK, _D), jnp.float32),
        pltpu.SemaphoreType.DMA,
    ],
    compiler_params=pltpu.CompilerParams(needs_layout_passes=False),
)
def _sc_prompts(table, gpos, gneg, lpos, lneg, tok,
                out_gp, out_gn, out_lp, out_ln, tokv, buf, sem):
    out_refs = (out_gp, out_gn, out_lp, out_ln)
    ctx_runs = (((gpos, 1),), ((gpos, 1), (gneg, 1 + _NPOS)),
                ((lpos, 1),), ((lpos, 1), (lneg, 1 + _NPOS)))
    wid = lax.axis_index("s") * _NC + lax.axis_index("c")
    iota = lax.iota(jnp.int32, _CHUNK)
    for s in range(_NSEG):
        nctx = _N_CTX_SEG[s]
        njobs = _LSEQ - nctx
        ntokw = 5 if s % 2 == 0 else 4
        lo = _WPS * s

        @pl.when((wid >= lo) & (wid < lo + _WPS))
        def _(s=s, nctx=nctx, njobs=njobs, ntokw=ntokw, lo=lo):
            k = wid - lo

            @pl.when(k < ntokw)
            def _():
                pltpu.sync_copy(tok, tokv)
                i = jnp.minimum(iota + k * _CHUNK, njobs - 1)
                pos = jnp.where(i < 1, 0, nctx + i)
                sidx = plsc.load_gather(tokv, [_LSEQ * s + pos])
                pltpu.async_copy(table.at[sidx], buf, sem).wait()
                pltpu.async_copy(buf, out_refs[s].at[pos], sem).wait()

            for t, (cref, dst0) in enumerate(ctx_runs[s]):
                @pl.when(k == ntokw + t)
                def _(cref=cref, dst0=dst0):
                    c = jnp.minimum(iota, _NPOS - 1)
                    pltpu.async_copy(cref.at[c], buf, sem).wait()
                    pltpu.async_copy(buf, out_refs[s].at[dst0 + c],
                                     sem).wait()


def _proj_body(cp_ref, w_ref, b_ref, out_ref):
    out_ref[...] = (
        jnp.dot(cp_ref[0], w_ref[0], preferred_element_type=jnp.float32)
        + b_ref[0]
    )[None]


_proj = pl.pallas_call(
    _proj_body,
    grid=(8,),
    in_specs=[
        pl.BlockSpec((1, _DEEP, _D), lambda l: (l, 0, 0)),
        pl.BlockSpec((1, _D, _PROJ), lambda l: (l, 0, 0)),
        pl.BlockSpec((1, 1, _PROJ), lambda l: (l, 0, 0)),
    ],
    out_specs=pl.BlockSpec((1, _DEEP, _PROJ), lambda l: (l, 0, 0)),
    out_shape=jax.ShapeDtypeStruct((8, _DEEP, _PROJ), jnp.float32),
)


def kernel(token_embedding, ctx_global_pos, ctx_global_neg, ctx_local_pos,
           ctx_local_neg, compound_prompts_text, proj_W, proj_b,
           tokens_global_pos, tokens_global_neg, tokens_local_pos,
           tokens_local_neg):
    tok_flat = jnp.concatenate([
        tokens_global_pos.reshape(-1), tokens_global_neg.reshape(-1),
        tokens_local_pos.reshape(-1), tokens_local_neg.reshape(-1),
    ])
    outs = _sc_prompts(
        token_embedding,
        ctx_global_pos.reshape(_NPOS, _D), ctx_global_neg.reshape(_NPOS, _D),
        ctx_local_pos.reshape(_NPOS, _D), ctx_local_neg.reshape(_NPOS, _D),
        tok_flat,
    )
    projected = _proj(compound_prompts_text, proj_W,
                      proj_b.reshape(8, 1, _PROJ))
    return (*(o.reshape(1, _LSEQ, _D) for o in outs), projected)


# speedup vs baseline: 1.0895x; 1.0842x over previous
import functools

import jax
import jax.numpy as jnp
from jax import lax
from jax.experimental import pallas as pl
from jax.experimental.pallas import tpu as pltpu
from jax.experimental.pallas import tpu_sc as plsc

_D = 768
_NPOS = 12
_DEEP = 4
_PROJ = 896
_LSEQ = 77
_NSEG = 4
_CHUNK = 16

_N_CTX_SEG = (_NPOS, 2 * _NPOS, _NPOS, 2 * _NPOS)
_WPS = 6
_NCHIP = _D // 128
_OUT_ROWS = _LSEQ * _NCHIP
_OUT_PAD = _OUT_ROWS + 2

_info = plsc.get_sparse_core_info()
_NC = _info.num_cores


@functools.partial(
    pl.kernel,
    mesh=plsc.VectorSubcoreMesh(core_axis_name="c", subcore_axis_name="s"),
    out_type=tuple(
        jax.ShapeDtypeStruct((_OUT_PAD, 128), jnp.float32)
        for _ in range(_NSEG)
    ),
    scratch_types=[
        pltpu.VMEM((_NSEG * _LSEQ,), jnp.int32),
        pltpu.VMEM((_CHUNK, _D), jnp.float32),
        pltpu.VMEM((_NPOS * _NCHIP, 128), jnp.float32),
        pltpu.SemaphoreType.DMA,
    ],
    compiler_params=pltpu.CompilerParams(needs_layout_passes=False),
)
def _sc_prompts(table, gpos, gneg, lpos, lneg, tok,
                out_gp, out_gn, out_lp, out_ln, tokv, buf, cbuf, sem):
    out_refs = (out_gp, out_gn, out_lp, out_ln)
    ctx_runs = (((gpos, 1),), ((gpos, 1), (gneg, 1 + _NPOS)),
                ((lpos, 1),), ((lpos, 1), (lneg, 1 + _NPOS)))
    wid = lax.axis_index("s") * _NC + lax.axis_index("c")
    iota = lax.iota(jnp.int32, _CHUNK)
    for s in range(_NSEG):
        nctx = _N_CTX_SEG[s]
        njobs = _LSEQ - nctx
        ntokw = 5 if s % 2 == 0 else 4
        lo = _WPS * s

        @pl.when((wid >= lo) & (wid < lo + _WPS))
        def _(s=s, nctx=nctx, njobs=njobs, ntokw=ntokw, lo=lo):
            k = wid - lo

            @pl.when(k < ntokw)
            def _():
                pltpu.sync_copy(tok, tokv)
                i = jnp.minimum(iota + k * _CHUNK, njobs - 1)
                pos = jnp.where(i < 1, 0, nctx + i)
                sidx = plsc.load_gather(tokv, [_LSEQ * s + pos])
                pltpu.async_copy(table.at[sidx], buf, sem).wait()
                puts = [
                    pltpu.async_copy(
                        buf.at[pl.ds(0, _CHUNK), pl.ds(128 * j, 128)],
                        out_refs[s].at[pos * _NCHIP + j], sem)
                    for j in range(_NCHIP)
                ]
                for p in puts:
                    p.wait()

            for t, (cref, dst0) in enumerate(ctx_runs[s]):
                @pl.when(k == ntokw + t)
                def _(cref=cref, dst0=dst0):
                    pltpu.sync_copy(cref, cbuf)
                    nrow = _NPOS * _NCHIP
                    offs = [*range(0, nrow - _CHUNK, _CHUNK), nrow - _CHUNK]
                    puts = [
                        pltpu.async_copy(
                            cbuf.at[pl.ds(off, _CHUNK)],
                            out_refs[s].at[dst0 * _NCHIP + off + iota], sem)
                        for off in offs
                    ]
                    for p in puts:
                        p.wait()


def _proj_body(cp_ref, w_ref, b_ref, out_ref):
    out_ref[...] = (
        jnp.dot(cp_ref[0], w_ref[0], preferred_element_type=jnp.float32)
        + b_ref[0]
    )[None]


_proj = pl.pallas_call(
    _proj_body,
    grid=(8,),
    in_specs=[
        pl.BlockSpec((1, _DEEP, _D), lambda l: (l, 0, 0)),
        pl.BlockSpec((1, _D, _PROJ), lambda l: (l, 0, 0)),
        pl.BlockSpec((1, 1, _PROJ), lambda l: (l, 0, 0)),
    ],
    out_specs=pl.BlockSpec((1, _DEEP, _PROJ), lambda l: (l, 0, 0)),
    out_shape=jax.ShapeDtypeStruct((8, _DEEP, _PROJ), jnp.float32),
)


def kernel(token_embedding, ctx_global_pos, ctx_global_neg, ctx_local_pos,
           ctx_local_neg, compound_prompts_text, proj_W, proj_b,
           tokens_global_pos, tokens_global_neg, tokens_local_pos,
           tokens_local_neg):
    tok_flat = jnp.concatenate([
        tokens_global_pos.reshape(-1), tokens_global_neg.reshape(-1),
        tokens_local_pos.reshape(-1), tokens_local_neg.reshape(-1),
    ])
    outs = _sc_prompts(
        token_embedding,
        ctx_global_pos.reshape(_NPOS * _NCHIP, 128),
        ctx_global_neg.reshape(_NPOS * _NCHIP, 128),
        ctx_local_pos.reshape(_NPOS * _NCHIP, 128),
        ctx_local_neg.reshape(_NPOS * _NCHIP, 128),
        tok_flat,
    )
    projected = _proj(compound_prompts_text, proj_W,
                      proj_b.reshape(8, 1, _PROJ))
    return (*(
        lax.slice(o, (0, 0), (_OUT_ROWS, 128)).reshape(1, _LSEQ, _D)
        for o in outs
    ), projected)


# speedup vs baseline: 1.0960x; 1.0060x over previous
import functools

import jax
import jax.numpy as jnp
from jax import lax
from jax.experimental import pallas as pl
from jax.experimental.pallas import tpu as pltpu
from jax.experimental.pallas import tpu_sc as plsc

_D = 768
_NPOS = 12
_DEEP = 4
_PROJ = 896
_LSEQ = 77
_NSEG = 4
_CHUNK = 16

_N_CTX_SEG = (_NPOS, 2 * _NPOS, _NPOS, 2 * _NPOS)
_WPS = 6
_NCHIP = _D // 128
_OUT_ROWS = _LSEQ * _NCHIP
_OUT_PAD = _OUT_ROWS + 2

_info = plsc.get_sparse_core_info()
_NC = _info.num_cores


@functools.partial(
    pl.kernel,
    mesh=plsc.VectorSubcoreMesh(core_axis_name="c", subcore_axis_name="s"),
    out_type=tuple(
        jax.ShapeDtypeStruct((_OUT_PAD, 128), jnp.float32)
        for _ in range(_NSEG)
    ),
    scratch_types=[
        pltpu.VMEM((_NSEG * _LSEQ,), jnp.int32),
        pltpu.VMEM((_CHUNK, _D), jnp.float32),
        pltpu.VMEM((_NPOS * _NCHIP, 128), jnp.float32),
        pltpu.SemaphoreType.DMA,
    ],
    compiler_params=pltpu.CompilerParams(
        needs_layout_passes=False, skip_device_barrier=True,
        disable_semaphore_checks=True),
)
def _sc_prompts(table, gpos, gneg, lpos, lneg, tok,
                out_gp, out_gn, out_lp, out_ln, tokv, buf, cbuf, sem):
    out_refs = (out_gp, out_gn, out_lp, out_ln)
    ctx_runs = (((gpos, 1),), ((gpos, 1), (gneg, 1 + _NPOS)),
                ((lpos, 1),), ((lpos, 1), (lneg, 1 + _NPOS)))
    wid = lax.axis_index("s") * _NC + lax.axis_index("c")
    iota = lax.iota(jnp.int32, _CHUNK)
    for s in range(_NSEG):
        nctx = _N_CTX_SEG[s]
        njobs = _LSEQ - nctx
        ntokw = 5 if s % 2 == 0 else 4
        lo = _WPS * s

        @pl.when((wid >= lo) & (wid < lo + _WPS))
        def _(s=s, nctx=nctx, njobs=njobs, ntokw=ntokw, lo=lo):
            k = wid - lo

            @pl.when(k < ntokw)
            def _():
                pltpu.sync_copy(tok, tokv)
                i = jnp.minimum(iota + k * _CHUNK, njobs - 1)
                pos = jnp.where(i < 1, 0, nctx + i)
                sidx = plsc.load_gather(tokv, [_LSEQ * s + pos])
                pltpu.async_copy(table.at[sidx], buf, sem).wait()
                puts = [
                    pltpu.async_copy(
                        buf.at[pl.ds(0, _CHUNK), pl.ds(128 * j, 128)],
                        out_refs[s].at[pos * _NCHIP + j], sem)
                    for j in range(_NCHIP)
                ]
                for p in puts:
                    p.wait()

            for t, (cref, dst0) in enumerate(ctx_runs[s]):
                @pl.when(k == ntokw + t)
                def _(cref=cref, dst0=dst0):
                    pltpu.sync_copy(cref, cbuf)
                    nrow = _NPOS * _NCHIP
                    offs = [*range(0, nrow - _CHUNK, _CHUNK), nrow - _CHUNK]
                    puts = [
                        pltpu.async_copy(
                            cbuf.at[pl.ds(off, _CHUNK)],
                            out_refs[s].at[dst0 * _NCHIP + off + iota], sem)
                        for off in offs
                    ]
                    for p in puts:
                        p.wait()


def _proj_body(cp_ref, w_ref, b_ref, out_ref):
    out_ref[...] = (
        jnp.dot(cp_ref[0], w_ref[0], preferred_element_type=jnp.float32)
        + b_ref[0]
    )[None]


_proj = pl.pallas_call(
    _proj_body,
    grid=(8,),
    in_specs=[
        pl.BlockSpec((1, _DEEP, _D), lambda l: (l, 0, 0)),
        pl.BlockSpec((1, _D, _PROJ), lambda l: (l, 0, 0)),
        pl.BlockSpec((1, 1, _PROJ), lambda l: (l, 0, 0)),
    ],
    out_specs=pl.BlockSpec((1, _DEEP, _PROJ), lambda l: (l, 0, 0)),
    out_shape=jax.ShapeDtypeStruct((8, _DEEP, _PROJ), jnp.float32),
)


def kernel(token_embedding, ctx_global_pos, ctx_global_neg, ctx_local_pos,
           ctx_local_neg, compound_prompts_text, proj_W, proj_b,
           tokens_global_pos, tokens_global_neg, tokens_local_pos,
           tokens_local_neg):
    tok_flat = jnp.concatenate([
        tokens_global_pos.reshape(-1), tokens_global_neg.reshape(-1),
        tokens_local_pos.reshape(-1), tokens_local_neg.reshape(-1),
    ])
    outs = _sc_prompts(
        token_embedding,
        ctx_global_pos.reshape(_NPOS * _NCHIP, 128),
        ctx_global_neg.reshape(_NPOS * _NCHIP, 128),
        ctx_local_pos.reshape(_NPOS * _NCHIP, 128),
        ctx_local_neg.reshape(_NPOS * _NCHIP, 128),
        tok_flat,
    )
    projected = _proj(compound_prompts_text, proj_W,
                      proj_b.reshape(8, 1, _PROJ))
    return (*(
        lax.slice(o, (0, 0), (_OUT_ROWS, 128)).reshape(1, _LSEQ, _D)
        for o in outs
    ), projected)


# speedup vs baseline: 1.1691x; 1.0667x over previous
import functools

import jax
import jax.numpy as jnp
from jax import lax
from jax.experimental import pallas as pl
from jax.experimental.pallas import tpu as pltpu
from jax.experimental.pallas import tpu_sc as plsc

_D = 768
_NPOS = 12
_DEEP = 4
_PROJ = 896
_LSEQ = 77
_NSEG = 4
_CHUNK = 16

_N_CTX_SEG = (_NPOS, 2 * _NPOS, _NPOS, 2 * _NPOS)
_WPS = 6
_NCHIP = _D // 128
_OUT_ROWS = _LSEQ * _NCHIP
_OUT_PAD = _OUT_ROWS + 2

_info = plsc.get_sparse_core_info()
_NC = _info.num_cores


@functools.partial(
    pl.kernel,
    mesh=plsc.VectorSubcoreMesh(core_axis_name="c", subcore_axis_name="s"),
    out_type=tuple(
        jax.ShapeDtypeStruct((_OUT_PAD, 128), jnp.float32)
        for _ in range(_NSEG)
    ),
    scratch_types=[
        pltpu.VMEM((_LSEQ,), jnp.int32),
        pltpu.VMEM((_CHUNK, _D), jnp.float32),
        pltpu.VMEM((_NPOS * _NCHIP, 128), jnp.float32),
        pltpu.SemaphoreType.DMA,
    ],
    compiler_params=pltpu.CompilerParams(
        needs_layout_passes=False, skip_device_barrier=True,
        disable_semaphore_checks=True),
)
def _sc_prompts(table, gpos, gneg, lpos, lneg, tok0, tok1, tok2, tok3,
                out_gp, out_gn, out_lp, out_ln, tokv, buf, cbuf, sem):
    out_refs = (out_gp, out_gn, out_lp, out_ln)
    tok_refs = (tok0, tok1, tok2, tok3)
    ctx_runs = (((gpos, 1),), ((gpos, 1), (gneg, 1 + _NPOS)),
                ((lpos, 1),), ((lpos, 1), (lneg, 1 + _NPOS)))
    wid = lax.axis_index("s") * _NC + lax.axis_index("c")
    iota = lax.iota(jnp.int32, _CHUNK)
    for s in range(_NSEG):
        nctx = _N_CTX_SEG[s]
        njobs = _LSEQ - nctx
        ntokw = 5 if s % 2 == 0 else 4
        lo = _WPS * s

        @pl.when((wid >= lo) & (wid < lo + _WPS))
        def _(s=s, nctx=nctx, njobs=njobs, ntokw=ntokw, lo=lo):
            k = wid - lo

            @pl.when(k < ntokw)
            def _():
                pltpu.sync_copy(tok_refs[s], tokv)
                i = jnp.minimum(iota + k * _CHUNK, njobs - 1)
                pos = jnp.where(i < 1, 0, nctx + i)
                sidx = plsc.load_gather(tokv, [pos])
                pltpu.async_copy(table.at[sidx], buf, sem).wait()
                puts = [
                    pltpu.async_copy(
                        buf.at[pl.ds(0, _CHUNK), pl.ds(128 * j, 128)],
                        out_refs[s].at[pos * _NCHIP + j], sem)
                    for j in range(_NCHIP)
                ]
                for p in puts:
                    p.wait()

            for t, (cref, dst0) in enumerate(ctx_runs[s]):
                @pl.when(k == ntokw + t)
                def _(cref=cref, dst0=dst0):
                    pltpu.sync_copy(cref, cbuf)
                    nrow = _NPOS * _NCHIP
                    offs = [*range(0, nrow - _CHUNK, _CHUNK), nrow - _CHUNK]
                    puts = [
                        pltpu.async_copy(
                            cbuf.at[pl.ds(off, _CHUNK)],
                            out_refs[s].at[dst0 * _NCHIP + off + iota], sem)
                        for off in offs
                    ]
                    for p in puts:
                        p.wait()


def _proj_body(cp_ref, w_ref, b_ref, out_ref):
    l = pl.program_id(0)
    out_ref[...] = (
        jnp.dot(cp_ref[0], w_ref[0], preferred_element_type=jnp.float32)
        + b_ref[pl.ds(l, 1)]
    )[None]


_proj = pl.pallas_call(
    _proj_body,
    grid=(8,),
    in_specs=[
        pl.BlockSpec((1, _DEEP, _D), lambda l: (l, 0, 0)),
        pl.BlockSpec((1, _D, _PROJ), lambda l: (l, 0, 0)),
        pl.BlockSpec((8, _PROJ), lambda l: (0, 0)),
    ],
    out_specs=pl.BlockSpec((1, _DEEP, _PROJ), lambda l: (l, 0, 0)),
    out_shape=jax.ShapeDtypeStruct((8, _DEEP, _PROJ), jnp.float32),
)


def kernel(token_embedding, ctx_global_pos, ctx_global_neg, ctx_local_pos,
           ctx_local_neg, compound_prompts_text, proj_W, proj_b,
           tokens_global_pos, tokens_global_neg, tokens_local_pos,
           tokens_local_neg):
    outs = _sc_prompts(
        token_embedding,
        ctx_global_pos.reshape(_NPOS * _NCHIP, 128),
        ctx_global_neg.reshape(_NPOS * _NCHIP, 128),
        ctx_local_pos.reshape(_NPOS * _NCHIP, 128),
        ctx_local_neg.reshape(_NPOS * _NCHIP, 128),
        tokens_global_pos.reshape(-1), tokens_global_neg.reshape(-1),
        tokens_local_pos.reshape(-1), tokens_local_neg.reshape(-1),
    )
    projected = _proj(compound_prompts_text, proj_W, proj_b)
    return (*(
        lax.slice(o, (0, 0), (_OUT_ROWS, 128)).reshape(1, _LSEQ, _D)
        for o in outs
    ), projected)
